# Initial kernel scaffold; baseline (speedup 1.0000x reference)
import functools

import jax
import jax.numpy as jnp
from jax import lax
from jax.experimental import pallas as pl
from jax.experimental.pallas import tpu as pltpu

E = 8
D = 768
F = 2048
T = 2048
FBLK = 1024
NF = F // FBLK


def _dot(a, b, **kw):
    return lax.dot_general(a, b, (((1,), (0,)), ((), ())), **kw)


def _router_coeff(x, gate_w, gate_b, e):
    logits = _dot(x, gate_w, precision=lax.Precision.HIGHEST) + gate_b
    col = lax.broadcasted_iota(jnp.int32, logits.shape, 1)
    m0 = jnp.max(logits, axis=1, keepdims=True)
    a0 = jnp.min(jnp.where(logits == m0, col, E), axis=1, keepdims=True)
    l2 = jnp.where(col == a0, -jnp.inf, logits)
    m1 = jnp.max(l2, axis=1, keepdims=True)
    a1 = jnp.min(jnp.where(l2 == m1, col, E), axis=1, keepdims=True)
    nw0 = 1.0 / (1.0 + jnp.exp(m1 - m0))
    nw1 = 1.0 - nw0
    return jnp.where(a0 == e, nw0, jnp.where(a1 == e, nw1, 0.0))


def _dense_body(x_ref, gw_ref, gb_ref, w1_ref, b1_ref, w2_ref, b2_ref, out_ref):
    e = pl.program_id(0)
    f = pl.program_id(1)
    x = x_ref[...]
    coeff = _router_coeff(x, gw_ref[...], gb_ref[...], e)

    xb = x.astype(jnp.bfloat16)
    w1 = w1_ref[0].astype(jnp.bfloat16)
    h = _dot(xb, w1, preferred_element_type=jnp.float32) + b1_ref[...]
    h = jax.nn.gelu(h)
    w2 = w2_ref[0].astype(jnp.bfloat16)
    y = _dot(h.astype(jnp.bfloat16), w2, preferred_element_type=jnp.float32)
    y = y + jnp.where(f == 0, 1.0, 0.0) * b2_ref[...]

    @pl.when((e == 0) & (f == 0))
    def _init():
        out_ref[...] = jnp.zeros_like(out_ref)

    out_ref[...] += y * coeff


def kernel(inputs, gate_w, gate_b, w1, b1, w2, b2):
    x = inputs.reshape(T, D)
    out2d = pl.pallas_call(
        _dense_body,
        grid=(E, NF),
        in_specs=[
            pl.BlockSpec((T, D), lambda e, f: (0, 0)),
            pl.BlockSpec((D, E), lambda e, f: (0, 0)),
            pl.BlockSpec((1, E), lambda e, f: (0, 0)),
            pl.BlockSpec((1, D, FBLK), lambda e, f: (e, 0, f)),
            pl.BlockSpec((1, FBLK), lambda e, f: (e, f)),
            pl.BlockSpec((1, FBLK, D), lambda e, f: (e, f, 0)),
            pl.BlockSpec((1, D), lambda e, f: (e, 0)),
        ],
        out_specs=pl.BlockSpec((T, D), lambda e, f: (0, 0)),
        out_shape=jax.ShapeDtypeStruct((T, D), jnp.float32),
        compiler_params=pltpu.CompilerParams(
            dimension_semantics=("arbitrary", "arbitrary")),
    )(x, gate_w, gate_b.reshape(1, E), w1, b1, w2, b2)
    return out2d.reshape(inputs.shape[:-1] + (D,))


# speedup vs baseline: 1.1439x; 1.1439x over previous
import functools

import jax
import jax.numpy as jnp
from jax import lax
from jax.experimental import pallas as pl
from jax.experimental.pallas import tpu as pltpu

E = 8
D = 768
F = 2048
T = 2048
FBLK = 1024
NF = F // FBLK


def _dot(a, b, **kw):
    return lax.dot_general(a, b, (((1,), (0,)), ((), ())), **kw)


def _router_coeff(x, gate_w, gate_b, e):
    logits = _dot(x.astype(jnp.bfloat16), gate_w.astype(jnp.bfloat16),
                  preferred_element_type=jnp.float32) + gate_b
    col = lax.broadcasted_iota(jnp.int32, logits.shape, 1)
    m0 = jnp.max(logits, axis=1, keepdims=True)
    a0 = jnp.min(jnp.where(logits == m0, col, E), axis=1, keepdims=True)
    l2 = jnp.where(col == a0, -jnp.inf, logits)
    m1 = jnp.max(l2, axis=1, keepdims=True)
    a1 = jnp.min(jnp.where(l2 == m1, col, E), axis=1, keepdims=True)
    nw0 = 1.0 / (1.0 + jnp.exp(m1 - m0))
    nw1 = 1.0 - nw0
    return jnp.where(a0 == e, nw0, jnp.where(a1 == e, nw1, 0.0))


def _dense_body(x_ref, gw_ref, gb_ref, w1_ref, b1_ref, w2_ref, b2_ref, out_ref):
    e = pl.program_id(0)
    f = pl.program_id(1)
    x = x_ref[...]
    coeff = _router_coeff(x, gw_ref[...], gb_ref[...], e)

    xb = x.astype(jnp.bfloat16)
    w1 = w1_ref[0].astype(jnp.bfloat16)
    h = _dot(xb, w1, preferred_element_type=jnp.float32) + b1_ref[0]
    h = jax.nn.gelu(h)
    w2 = w2_ref[0].astype(jnp.bfloat16)
    y = _dot(h.astype(jnp.bfloat16), w2, preferred_element_type=jnp.float32)
    y = y + jnp.where(f == 0, 1.0, 0.0) * b2_ref[0]

    @pl.when((e == 0) & (f == 0))
    def _init():
        out_ref[...] = jnp.zeros_like(out_ref)

    out_ref[...] += y * coeff


def kernel(inputs, gate_w, gate_b, w1, b1, w2, b2):
    x = inputs.reshape(T, D)
    out2d = pl.pallas_call(
        _dense_body,
        grid=(E, NF),
        in_specs=[
            pl.BlockSpec((T, D), lambda e, f: (0, 0)),
            pl.BlockSpec((D, E), lambda e, f: (0, 0)),
            pl.BlockSpec((1, E), lambda e, f: (0, 0)),
            pl.BlockSpec((1, D, FBLK), lambda e, f: (e, 0, f)),
            pl.BlockSpec((1, 1, FBLK), lambda e, f: (e, 0, f)),
            pl.BlockSpec((1, FBLK, D), lambda e, f: (e, f, 0)),
            pl.BlockSpec((1, 1, D), lambda e, f: (e, 0, 0)),
        ],
        out_specs=pl.BlockSpec((T, D), lambda e, f: (0, 0)),
        out_shape=jax.ShapeDtypeStruct((T, D), jnp.float32),
        compiler_params=pltpu.CompilerParams(
            dimension_semantics=("arbitrary", "arbitrary")),
    )(x, gate_w, gate_b.reshape(1, E), w1, b1.reshape(E, 1, F), w2,
      b2.reshape(E, 1, D))
    return out2d.reshape(inputs.shape[:-1] + (D,))


# speedup vs baseline: 1.3515x; 1.1815x over previous
import functools

import jax
import jax.numpy as jnp
from jax import lax
from jax.experimental import pallas as pl
from jax.experimental.pallas import tpu as pltpu
from jax.experimental.pallas import tpu_sc as plsc

E = 8
D = 768
F = 2048
T = 2048
BT = 128
RPAD = 2 * T + E * BT
NB = RPAD // BT
NTILES = 32
TPW = T // NTILES


def _dot(a, b, **kw):
    return lax.dot_general(a, b, (((1,), (0,)), ((), ())), **kw)


def _router_body(x_ref, gw_ref, gb_ref, pos_ref, nw_ref, sp_ref):
    x = x_ref[...]
    logits = _dot(x.astype(jnp.bfloat16), gw_ref[...].astype(jnp.bfloat16),
                  preferred_element_type=jnp.float32) + gb_ref[...]
    col = lax.broadcasted_iota(jnp.int32, logits.shape, 1)
    m0 = jnp.max(logits, axis=1, keepdims=True)
    a0 = jnp.min(jnp.where(logits == m0, col, E), axis=1, keepdims=True)
    l2 = jnp.where(col == a0, -jnp.inf, logits)
    m1 = jnp.max(l2, axis=1, keepdims=True)
    a1 = jnp.min(jnp.where(l2 == m1, col, E), axis=1, keepdims=True)
    nw0 = 1.0 / (1.0 + jnp.exp(m1 - m0))
    nw1 = 1.0 - nw0

    nw_ref[0:1, :] = nw0.T
    nw_ref[1:2, :] = nw1.T

    oh0 = (col == a0).astype(jnp.int32)
    oh1 = (col == a1).astype(jnp.int32)
    oh = jnp.concatenate([oh0.T, oh1.T], axis=1)
    cum = oh
    s = 1
    while s < 2 * T:
        cum = cum + jnp.concatenate(
            [jnp.zeros((E, s), jnp.int32), cum[:, : 2 * T - s]], axis=1)
        s *= 2
    counts = cum[:, 2 * T - 1 : 2 * T]
    pc = ((counts + BT - 1) // BT) * BT
    inc = pc
    s = 1
    while s < E:
        inc = inc + jnp.concatenate(
            [jnp.zeros((s, 1), jnp.int32), inc[: E - s, :]], axis=0)
        s *= 2
    sexc = inc - pc
    pos_all = jnp.sum(oh * (cum + sexc - 1), axis=0, keepdims=True)
    pos_ref[0:1, :] = pos_all[:, :T]
    pos_ref[1:2, :] = pos_all[:, T:]

    lane = lax.broadcasted_iota(jnp.int32, (1, 64), 1)
    g = jnp.sum((lane * BT >= inc).astype(jnp.int32), axis=0, keepdims=True)
    g = jnp.minimum(g, E - 1)
    used = inc[E - 1 : E, 0:1] // BT
    sp_ref[...] = jnp.where(lane == NB, jnp.broadcast_to(used, (1, 64)), g)


def _router(x, gate_w, gate_b):
    return pl.pallas_call(
        _router_body,
        out_shape=[
            jax.ShapeDtypeStruct((2, T), jnp.int32),
            jax.ShapeDtypeStruct((2, T), jnp.float32),
            jax.ShapeDtypeStruct((1, 64), jnp.int32),
        ],
    )(x, gate_w, gate_b)


@functools.lru_cache(maxsize=None)
def _sc_dispatch_kernel():
    mesh = plsc.VectorSubcoreMesh(core_axis_name="c", subcore_axis_name="s")

    @functools.partial(
        pl.kernel,
        out_type=jax.ShapeDtypeStruct((RPAD, D), jnp.float32),
        mesh=mesh,
        scratch_types=[
            pltpu.VMEM((TPW,), jnp.int32),
            pltpu.VMEM((TPW,), jnp.int32),
            pltpu.VMEM((TPW, D), jnp.float32),
            pltpu.SemaphoreType.DMA,
        ],
    )
    def _sc_dispatch(x_hbm, pos_hbm, xs_hbm, idx0_v, idx1_v, rows_v, sem):
        wid = lax.axis_index("s") * 2 + lax.axis_index("c")
        base = wid * TPW
        pltpu.sync_copy(pos_hbm.at[0, pl.ds(base, TPW)], idx0_v)
        pltpu.sync_copy(pos_hbm.at[1, pl.ds(base, TPW)], idx1_v)
        pltpu.sync_copy(x_hbm.at[pl.ds(base, TPW), :], rows_v)
        c0 = pltpu.async_copy(rows_v, xs_hbm.at[idx0_v], sem)
        c1 = pltpu.async_copy(rows_v, xs_hbm.at[idx1_v], sem)
        c0.wait()
        c1.wait()

    return _sc_dispatch


def _mlp_body(sp_ref, xs_ref, w1_ref, b1_ref, w2_ref, b2_ref, y_ref):
    b = pl.program_id(0)

    @pl.when(b < sp_ref[NB])
    def _():
        xb = xs_ref[...].astype(jnp.bfloat16)
        w1 = w1_ref[0].astype(jnp.bfloat16)
        h = _dot(xb, w1, preferred_element_type=jnp.float32) + b1_ref[0]
        h = jax.nn.gelu(h)
        w2 = w2_ref[0].astype(jnp.bfloat16)
        y_ref[...] = (
            _dot(h.astype(jnp.bfloat16), w2, preferred_element_type=jnp.float32)
            + b2_ref[0])


def _mlp(spinfo, xs, w1, b1, w2, b2):
    grid_spec = pltpu.PrefetchScalarGridSpec(
        num_scalar_prefetch=1,
        grid=(NB,),
        in_specs=[
            pl.BlockSpec((BT, D), lambda b, sp: (b, 0)),
            pl.BlockSpec((1, D, F), lambda b, sp: (sp[b], 0, 0)),
            pl.BlockSpec((1, 1, F), lambda b, sp: (sp[b], 0, 0)),
            pl.BlockSpec((1, F, D), lambda b, sp: (sp[b], 0, 0)),
            pl.BlockSpec((1, 1, D), lambda b, sp: (sp[b], 0, 0)),
        ],
        out_specs=pl.BlockSpec((BT, D), lambda b, sp: (b, 0)),
    )
    return pl.pallas_call(
        _mlp_body,
        grid_spec=grid_spec,
        out_shape=jax.ShapeDtypeStruct((RPAD, D), jnp.float32),
        compiler_params=pltpu.CompilerParams(
            dimension_semantics=("arbitrary",)),
    )(spinfo, xs, w1, b1, w2, b2)


@functools.lru_cache(maxsize=None)
def _sc_combine_kernel():
    mesh = plsc.VectorSubcoreMesh(core_axis_name="c", subcore_axis_name="s")

    @functools.partial(
        pl.kernel,
        out_type=jax.ShapeDtypeStruct((T, D), jnp.float32),
        mesh=mesh,
        scratch_types=[
            pltpu.VMEM((TPW,), jnp.int32),
            pltpu.VMEM((TPW,), jnp.int32),
            pltpu.VMEM((TPW,), jnp.float32),
            pltpu.VMEM((TPW,), jnp.float32),
            pltpu.VMEM((TPW, D), jnp.float32),
            pltpu.VMEM((TPW, D), jnp.float32),
            pltpu.SemaphoreType.DMA,
        ],
    )
    def _sc_combine(y_hbm, pos_hbm, nw_hbm, out_hbm,
                    idx0_v, idx1_v, w0_v, w1_v, buf0, buf1, sem):
        wid = lax.axis_index("s") * 2 + lax.axis_index("c")
        base = wid * TPW
        pltpu.sync_copy(pos_hbm.at[0, pl.ds(base, TPW)], idx0_v)
        pltpu.sync_copy(pos_hbm.at[1, pl.ds(base, TPW)], idx1_v)
        pltpu.sync_copy(nw_hbm.at[0, pl.ds(base, TPW)], w0_v)
        pltpu.sync_copy(nw_hbm.at[1, pl.ds(base, TPW)], w1_v)
        c0 = pltpu.async_copy(y_hbm.at[idx0_v], buf0, sem)
        c1 = pltpu.async_copy(y_hbm.at[idx1_v], buf1, sem)
        c0.wait()
        c1.wait()

        def body(c, carry):
            wv0 = w0_v[pl.ds(c * 16, 16)]
            wv1 = w1_v[pl.ds(c * 16, 16)]
            for r in range(16):
                j = c * 16 + r
                w0 = wv0[r]
                w1 = wv1[r]
                for k in range(D // 16):
                    sl = pl.ds(k * 16, 16)
                    buf0[j, sl] = buf0[j, sl] * w0 + buf1[j, sl] * w1
            return carry

        lax.fori_loop(0, TPW // 16, body, 0)
        pltpu.sync_copy(buf0, out_hbm.at[pl.ds(base, TPW), :])

    return _sc_combine


def kernel(inputs, gate_w, gate_b, w1, b1, w2, b2):
    x = inputs.reshape(T, D)
    pos, nw, spinfo = _router(x, gate_w, gate_b.reshape(1, E))
    xs = _sc_dispatch_kernel()(x, pos)
    y = _mlp(spinfo.reshape(64), xs, w1, b1.reshape(E, 1, F), w2,
             b2.reshape(E, 1, D))
    out = _sc_combine_kernel()(y, pos, nw)
    return out.reshape(inputs.shape[:-1] + (D,))


# speedup vs baseline: 1.4647x; 1.0838x over previous
import functools

import jax
import jax.numpy as jnp
from jax import lax
from jax.experimental import pallas as pl
from jax.experimental.pallas import tpu as pltpu
from jax.experimental.pallas import tpu_sc as plsc

E = 8
D = 768
F = 2048
T = 2048
BT = 256
RPAD = 2 * T + E * BT
NB = RPAD // BT
NTILES = 32
TPW = T // NTILES


def _dot(a, b, **kw):
    return lax.dot_general(a, b, (((1,), (0,)), ((), ())), **kw)


def _router_body(x_ref, gw_ref, gb_ref, pos_ref, nw_ref, sp_ref):
    x = x_ref[...]
    logits = _dot(x.astype(jnp.bfloat16), gw_ref[...].astype(jnp.bfloat16),
                  preferred_element_type=jnp.float32) + gb_ref[...]
    col = lax.broadcasted_iota(jnp.int32, logits.shape, 1)
    m0 = jnp.max(logits, axis=1, keepdims=True)
    a0 = jnp.min(jnp.where(logits == m0, col, E), axis=1, keepdims=True)
    l2 = jnp.where(col == a0, -jnp.inf, logits)
    m1 = jnp.max(l2, axis=1, keepdims=True)
    a1 = jnp.min(jnp.where(l2 == m1, col, E), axis=1, keepdims=True)
    nw0 = 1.0 / (1.0 + jnp.exp(m1 - m0))
    nw1 = 1.0 - nw0

    nw_ref[0:1, :] = nw0.T
    nw_ref[1:2, :] = nw1.T

    oh0 = (col == a0).astype(jnp.int32)
    oh1 = (col == a1).astype(jnp.int32)
    oh = jnp.concatenate([oh0.T, oh1.T], axis=1)
    cum = oh
    s = 1
    while s < 2 * T:
        cum = cum + jnp.concatenate(
            [jnp.zeros((E, s), jnp.int32), cum[:, : 2 * T - s]], axis=1)
        s *= 2
    counts = cum[:, 2 * T - 1 : 2 * T]
    pc = ((counts + BT - 1) // BT) * BT
    inc = pc
    s = 1
    while s < E:
        inc = inc + jnp.concatenate(
            [jnp.zeros((s, 1), jnp.int32), inc[: E - s, :]], axis=0)
        s *= 2
    sexc = inc - pc
    pos_all = jnp.sum(oh * (cum + sexc - 1), axis=0, keepdims=True)
    pos_ref[0:1, :] = pos_all[:, :T]
    pos_ref[1:2, :] = pos_all[:, T:]

    lane = lax.broadcasted_iota(jnp.int32, (1, 64), 1)
    g = jnp.sum((lane * BT >= inc).astype(jnp.int32), axis=0, keepdims=True)
    g = jnp.minimum(g, E - 1)
    used = inc[E - 1 : E, 0:1] // BT
    sp_ref[...] = jnp.where(lane == NB, jnp.broadcast_to(used, (1, 64)), g)


def _router(x, gate_w, gate_b):
    return pl.pallas_call(
        _router_body,
        out_shape=[
            jax.ShapeDtypeStruct((2, T), jnp.int32),
            jax.ShapeDtypeStruct((2, T), jnp.float32),
            jax.ShapeDtypeStruct((1, 64), jnp.int32),
        ],
    )(x, gate_w, gate_b)


@functools.lru_cache(maxsize=None)
def _sc_dispatch_kernel():
    mesh = plsc.VectorSubcoreMesh(core_axis_name="c", subcore_axis_name="s")

    @functools.partial(
        pl.kernel,
        out_type=jax.ShapeDtypeStruct((RPAD, D), jnp.float32),
        mesh=mesh,
        scratch_types=[
            pltpu.VMEM((TPW,), jnp.int32),
            pltpu.VMEM((TPW,), jnp.int32),
            pltpu.VMEM((TPW, D), jnp.float32),
            pltpu.SemaphoreType.DMA,
        ],
    )
    def _sc_dispatch(x_hbm, pos_hbm, xs_hbm, idx0_v, idx1_v, rows_v, sem):
        wid = lax.axis_index("s") * 2 + lax.axis_index("c")
        base = wid * TPW
        pltpu.sync_copy(pos_hbm.at[0, pl.ds(base, TPW)], idx0_v)
        pltpu.sync_copy(pos_hbm.at[1, pl.ds(base, TPW)], idx1_v)
        pltpu.sync_copy(x_hbm.at[pl.ds(base, TPW), :], rows_v)
        c0 = pltpu.async_copy(rows_v, xs_hbm.at[idx0_v], sem)
        c1 = pltpu.async_copy(rows_v, xs_hbm.at[idx1_v], sem)
        c0.wait()
        c1.wait()

    return _sc_dispatch


def _mlp_body(sp_ref, xs_ref, w1_ref, b1_ref, w2_ref, b2_ref, y_ref):
    b = pl.program_id(0)

    @pl.when(b < sp_ref[NB])
    def _():
        xb = xs_ref[...].astype(jnp.bfloat16)
        w1 = w1_ref[0].astype(jnp.bfloat16)
        h = _dot(xb, w1, preferred_element_type=jnp.float32) + b1_ref[0]
        h = jax.nn.gelu(h)
        w2 = w2_ref[0].astype(jnp.bfloat16)
        y_ref[...] = (
            _dot(h.astype(jnp.bfloat16), w2, preferred_element_type=jnp.float32)
            + b2_ref[0])


def _mlp(spinfo, xs, w1, b1, w2, b2):
    grid_spec = pltpu.PrefetchScalarGridSpec(
        num_scalar_prefetch=1,
        grid=(NB,),
        in_specs=[
            pl.BlockSpec((BT, D), lambda b, sp: (b, 0)),
            pl.BlockSpec((1, D, F), lambda b, sp: (sp[b], 0, 0)),
            pl.BlockSpec((1, 1, F), lambda b, sp: (sp[b], 0, 0)),
            pl.BlockSpec((1, F, D), lambda b, sp: (sp[b], 0, 0)),
            pl.BlockSpec((1, 1, D), lambda b, sp: (sp[b], 0, 0)),
        ],
        out_specs=pl.BlockSpec((BT, D), lambda b, sp: (b, 0)),
    )
    return pl.pallas_call(
        _mlp_body,
        grid_spec=grid_spec,
        out_shape=jax.ShapeDtypeStruct((RPAD, D), jnp.float32),
        compiler_params=pltpu.CompilerParams(
            dimension_semantics=("arbitrary",)),
    )(spinfo, xs, w1, b1, w2, b2)


@functools.lru_cache(maxsize=None)
def _sc_combine_kernel():
    mesh = plsc.VectorSubcoreMesh(core_axis_name="c", subcore_axis_name="s")

    @functools.partial(
        pl.kernel,
        out_type=jax.ShapeDtypeStruct((T, D), jnp.float32),
        mesh=mesh,
        scratch_types=[
            pltpu.VMEM((TPW,), jnp.int32),
            pltpu.VMEM((TPW,), jnp.int32),
            pltpu.VMEM((TPW,), jnp.float32),
            pltpu.VMEM((TPW,), jnp.float32),
            pltpu.VMEM((TPW, D), jnp.float32),
            pltpu.VMEM((TPW, D), jnp.float32),
            pltpu.SemaphoreType.DMA,
        ],
    )
    def _sc_combine(y_hbm, pos_hbm, nw_hbm, out_hbm,
                    idx0_v, idx1_v, w0_v, w1_v, buf0, buf1, sem):
        wid = lax.axis_index("s") * 2 + lax.axis_index("c")
        base = wid * TPW
        pltpu.sync_copy(pos_hbm.at[0, pl.ds(base, TPW)], idx0_v)
        pltpu.sync_copy(pos_hbm.at[1, pl.ds(base, TPW)], idx1_v)
        pltpu.sync_copy(nw_hbm.at[0, pl.ds(base, TPW)], w0_v)
        pltpu.sync_copy(nw_hbm.at[1, pl.ds(base, TPW)], w1_v)
        c0 = pltpu.async_copy(y_hbm.at[idx0_v], buf0, sem)
        c1 = pltpu.async_copy(y_hbm.at[idx1_v], buf1, sem)
        c0.wait()
        c1.wait()

        def body(c, carry):
            wv0 = w0_v[pl.ds(c * 16, 16)]
            wv1 = w1_v[pl.ds(c * 16, 16)]
            for r in range(16):
                j = c * 16 + r
                w0 = wv0[r]
                w1 = wv1[r]
                for k in range(D // 16):
                    sl = pl.ds(k * 16, 16)
                    buf0[j, sl] = buf0[j, sl] * w0 + buf1[j, sl] * w1
            return carry

        lax.fori_loop(0, TPW // 16, body, 0)
        pltpu.sync_copy(buf0, out_hbm.at[pl.ds(base, TPW), :])

    return _sc_combine


def kernel(inputs, gate_w, gate_b, w1, b1, w2, b2):
    x = inputs.reshape(T, D)
    pos, nw, spinfo = _router(x, gate_w, gate_b.reshape(1, E))
    xs = _sc_dispatch_kernel()(x, pos)
    y = _mlp(spinfo.reshape(64), xs, w1, b1.reshape(E, 1, F), w2,
             b2.reshape(E, 1, D))
    out = _sc_combine_kernel()(y, pos, nw)
    return out.reshape(inputs.shape[:-1] + (D,))


# speedup vs baseline: 1.5880x; 1.0841x over previous
import functools

import jax
import jax.numpy as jnp
from jax import lax
from jax.experimental import pallas as pl
from jax.experimental.pallas import tpu as pltpu
from jax.experimental.pallas import tpu_sc as plsc

E = 8
D = 768
F = 2048
T = 2048
BT = 512
RPAD = 2 * T + E * BT
NB = RPAD // BT
NTILES = 32
TPW = T // NTILES


def _dot(a, b, **kw):
    return lax.dot_general(a, b, (((1,), (0,)), ((), ())), **kw)


def _router_body(x_ref, gw_ref, gb_ref, pos_ref, nw_ref, sp_ref):
    x = x_ref[...]
    logits = _dot(x.astype(jnp.bfloat16), gw_ref[...].astype(jnp.bfloat16),
                  preferred_element_type=jnp.float32) + gb_ref[...]
    col = lax.broadcasted_iota(jnp.int32, logits.shape, 1)
    m0 = jnp.max(logits, axis=1, keepdims=True)
    a0 = jnp.min(jnp.where(logits == m0, col, E), axis=1, keepdims=True)
    l2 = jnp.where(col == a0, -jnp.inf, logits)
    m1 = jnp.max(l2, axis=1, keepdims=True)
    a1 = jnp.min(jnp.where(l2 == m1, col, E), axis=1, keepdims=True)
    nw0 = 1.0 / (1.0 + jnp.exp(m1 - m0))
    nw1 = 1.0 - nw0

    nw_ref[0:1, :] = nw0.T
    nw_ref[1:2, :] = nw1.T

    oh0 = (col == a0).astype(jnp.int32)
    oh1 = (col == a1).astype(jnp.int32)
    oh = jnp.concatenate([oh0.T, oh1.T], axis=1)
    cum = oh
    s = 1
    while s < 2 * T:
        cum = cum + jnp.concatenate(
            [jnp.zeros((E, s), jnp.int32), cum[:, : 2 * T - s]], axis=1)
        s *= 2
    counts = cum[:, 2 * T - 1 : 2 * T]
    pc = ((counts + BT - 1) // BT) * BT
    inc = pc
    s = 1
    while s < E:
        inc = inc + jnp.concatenate(
            [jnp.zeros((s, 1), jnp.int32), inc[: E - s, :]], axis=0)
        s *= 2
    sexc = inc - pc
    pos_all = jnp.sum(oh * (cum + sexc - 1), axis=0, keepdims=True)
    pos_ref[0:1, :] = pos_all[:, :T]
    pos_ref[1:2, :] = pos_all[:, T:]

    lane = lax.broadcasted_iota(jnp.int32, (1, 64), 1)
    g = jnp.sum((lane * BT >= inc).astype(jnp.int32), axis=0, keepdims=True)
    g = jnp.minimum(g, E - 1)
    used = inc[E - 1 : E, 0:1] // BT
    sp_ref[...] = jnp.where(lane == NB, jnp.broadcast_to(used, (1, 64)), g)


def _router(x, gate_w, gate_b):
    return pl.pallas_call(
        _router_body,
        out_shape=[
            jax.ShapeDtypeStruct((2, T), jnp.int32),
            jax.ShapeDtypeStruct((2, T), jnp.float32),
            jax.ShapeDtypeStruct((1, 64), jnp.int32),
        ],
    )(x, gate_w, gate_b)


@functools.lru_cache(maxsize=None)
def _sc_dispatch_kernel():
    mesh = plsc.VectorSubcoreMesh(core_axis_name="c", subcore_axis_name="s")

    @functools.partial(
        pl.kernel,
        out_type=jax.ShapeDtypeStruct((RPAD, D), jnp.float32),
        mesh=mesh,
        scratch_types=[
            pltpu.VMEM((TPW,), jnp.int32),
            pltpu.VMEM((TPW,), jnp.int32),
            pltpu.VMEM((TPW, D), jnp.float32),
            pltpu.SemaphoreType.DMA,
        ],
    )
    def _sc_dispatch(x_hbm, pos_hbm, xs_hbm, idx0_v, idx1_v, rows_v, sem):
        wid = lax.axis_index("s") * 2 + lax.axis_index("c")
        base = wid * TPW
        pltpu.sync_copy(pos_hbm.at[0, pl.ds(base, TPW)], idx0_v)
        pltpu.sync_copy(pos_hbm.at[1, pl.ds(base, TPW)], idx1_v)
        pltpu.sync_copy(x_hbm.at[pl.ds(base, TPW), :], rows_v)
        c0 = pltpu.async_copy(rows_v, xs_hbm.at[idx0_v], sem)
        c1 = pltpu.async_copy(rows_v, xs_hbm.at[idx1_v], sem)
        c0.wait()
        c1.wait()

    return _sc_dispatch


def _mlp_body(sp_ref, xs_ref, w1_ref, b1_ref, w2_ref, b2_ref, y_ref):
    b = pl.program_id(0)

    @pl.when(b < sp_ref[NB])
    def _():
        xb = xs_ref[...].astype(jnp.bfloat16)
        w1 = w1_ref[0].astype(jnp.bfloat16)
        h = _dot(xb, w1, preferred_element_type=jnp.float32) + b1_ref[0]
        h = jax.nn.gelu(h)
        w2 = w2_ref[0].astype(jnp.bfloat16)
        y_ref[...] = (
            _dot(h.astype(jnp.bfloat16), w2, preferred_element_type=jnp.float32)
            + b2_ref[0])


def _mlp(spinfo, xs, w1, b1, w2, b2):
    grid_spec = pltpu.PrefetchScalarGridSpec(
        num_scalar_prefetch=1,
        grid=(NB,),
        in_specs=[
            pl.BlockSpec((BT, D), lambda b, sp: (b, 0)),
            pl.BlockSpec((1, D, F), lambda b, sp: (sp[b], 0, 0)),
            pl.BlockSpec((1, 1, F), lambda b, sp: (sp[b], 0, 0)),
            pl.BlockSpec((1, F, D), lambda b, sp: (sp[b], 0, 0)),
            pl.BlockSpec((1, 1, D), lambda b, sp: (sp[b], 0, 0)),
        ],
        out_specs=pl.BlockSpec((BT, D), lambda b, sp: (b, 0)),
    )
    return pl.pallas_call(
        _mlp_body,
        grid_spec=grid_spec,
        out_shape=jax.ShapeDtypeStruct((RPAD, D), jnp.float32),
        compiler_params=pltpu.CompilerParams(
            dimension_semantics=("arbitrary",)),
    )(spinfo, xs, w1, b1, w2, b2)


@functools.lru_cache(maxsize=None)
def _sc_combine_kernel():
    mesh = plsc.VectorSubcoreMesh(core_axis_name="c", subcore_axis_name="s")

    @functools.partial(
        pl.kernel,
        out_type=jax.ShapeDtypeStruct((T, D), jnp.float32),
        mesh=mesh,
        scratch_types=[
            pltpu.VMEM((TPW,), jnp.int32),
            pltpu.VMEM((TPW,), jnp.int32),
            pltpu.VMEM((TPW,), jnp.float32),
            pltpu.VMEM((TPW,), jnp.float32),
            pltpu.VMEM((TPW, D), jnp.float32),
            pltpu.VMEM((TPW, D), jnp.float32),
            pltpu.SemaphoreType.DMA,
        ],
    )
    def _sc_combine(y_hbm, pos_hbm, nw_hbm, out_hbm,
                    idx0_v, idx1_v, w0_v, w1_v, buf0, buf1, sem):
        wid = lax.axis_index("s") * 2 + lax.axis_index("c")
        base = wid * TPW
        pltpu.sync_copy(pos_hbm.at[0, pl.ds(base, TPW)], idx0_v)
        pltpu.sync_copy(pos_hbm.at[1, pl.ds(base, TPW)], idx1_v)
        pltpu.sync_copy(nw_hbm.at[0, pl.ds(base, TPW)], w0_v)
        pltpu.sync_copy(nw_hbm.at[1, pl.ds(base, TPW)], w1_v)
        c0 = pltpu.async_copy(y_hbm.at[idx0_v], buf0, sem)
        c1 = pltpu.async_copy(y_hbm.at[idx1_v], buf1, sem)
        c0.wait()
        c1.wait()

        def body(c, carry):
            wv0 = w0_v[pl.ds(c * 16, 16)]
            wv1 = w1_v[pl.ds(c * 16, 16)]
            for r in range(16):
                j = c * 16 + r
                w0 = wv0[r]
                w1 = wv1[r]
                for k in range(D // 16):
                    sl = pl.ds(k * 16, 16)
                    buf0[j, sl] = buf0[j, sl] * w0 + buf1[j, sl] * w1
            return carry

        lax.fori_loop(0, TPW // 16, body, 0)
        pltpu.sync_copy(buf0, out_hbm.at[pl.ds(base, TPW), :])

    return _sc_combine


def kernel(inputs, gate_w, gate_b, w1, b1, w2, b2):
    x = inputs.reshape(T, D)
    pos, nw, spinfo = _router(x, gate_w, gate_b.reshape(1, E))
    xs = _sc_dispatch_kernel()(x, pos)
    y = _mlp(spinfo.reshape(64), xs, w1, b1.reshape(E, 1, F), w2,
             b2.reshape(E, 1, D))
    out = _sc_combine_kernel()(y, pos, nw)
    return out.reshape(inputs.shape[:-1] + (D,))
